# baseline (speedup 1.0000x reference)
import math

import jax
import jax.numpy as jnp
from jax import lax
from jax.experimental import pallas as pl
from jax.experimental.pallas import tpu as pltpu
from jax.experimental.pallas import tpu_sc as plsc

_N_NODES = 100_000
_N_EDGES = 6_400_000
_FACTOR = 1.0 / math.sqrt(64.0)

_NC = 2
_NS = 16
_NW = _NC * _NS
_EPW = _N_EDGES // _NW
_CHUNK = 4_000
_NCHUNK = _EPW // _CHUNK
_LANES = 16
_N_ACC = 100_000
_UNROLL = 10
_ZUNROLL = 10


def _sc_scatter_phase(idx_hbm, val_hbm, out_hbm,
                      idxb0, valb0, idxb1, valb1, acc, sem0, sem1):
    wid = lax.axis_index("s") * _NC + lax.axis_index("c")
    base = wid * _EPW

    idxbs = (idxb0, idxb1)
    valbs = (valb0, valb1)
    sems = (sem0, sem1)

    def start(c, b):
        off = base + c * _CHUNK
        pltpu.async_copy(idx_hbm.at[pl.ds(off, _CHUNK)], idxbs[b], sems[b])
        pltpu.async_copy(val_hbm.at[pl.ds(off, _CHUNK)], valbs[b], sems[b])

    def wait(b):
        pltpu.make_async_copy(idx_hbm.at[pl.ds(0, _CHUNK)], idxbs[b],
                              sems[b]).wait()
        pltpu.make_async_copy(val_hbm.at[pl.ds(0, _CHUNK)], valbs[b],
                              sems[b]).wait()

    start(0, 0)

    zeros = jnp.zeros((_LANES,), jnp.float32)

    def zbody(i, carry):
        for u in range(_ZUNROLL):
            acc[pl.ds((i * _ZUNROLL + u) * _LANES, _LANES)] = zeros
        return carry

    lax.fori_loop(0, _N_ACC // (_LANES * _ZUNROLL), zbody, 0)

    def process(b):
        ib, vb = idxbs[b], valbs[b]

        def sbody(j, carry):
            for u in range(_UNROLL):
                o = (j * _UNROLL + u) * _LANES
                iv = ib[pl.ds(o, _LANES)]
                vv = vb[pl.ds(o, _LANES)]
                plsc.addupdate_scatter(acc, [iv], vv)
            return carry

        lax.fori_loop(0, _CHUNK // (_LANES * _UNROLL), sbody, 0)

    def cbody(s, carry):
        c0 = 2 * s
        start(c0 + 1, 1)
        wait(0)
        process(0)
        @pl.when(c0 + 2 < _NCHUNK)
        def _():
            start(c0 + 2, 0)
        wait(1)
        process(1)
        return carry

    lax.fori_loop(0, _NCHUNK // 2, cbody, 0)

    pltpu.sync_copy(acc, out_hbm.at[wid])


def _tc_reduce_kernel(p_ref, o_ref):
    o_ref[...] = jnp.sum(p_ref[...], axis=0) * _FACTOR


@jax.jit
def kernel(edge_features, edge_index, atom_types):
    del atom_types
    idx_flat = edge_index.reshape(2 * _N_EDGES)
    val = edge_features.reshape(_N_EDGES)

    mesh = plsc.VectorSubcoreMesh(
        core_axis_name="c", subcore_axis_name="s",
        num_cores=_NC, num_subcores=_NS,
    )
    partials = pl.kernel(
        _sc_scatter_phase,
        out_type=jax.ShapeDtypeStruct((_NW, _N_ACC), jnp.float32),
        mesh=mesh,
        compiler_params=pltpu.CompilerParams(needs_layout_passes=False),
        scratch_types=[
            pltpu.VMEM((_CHUNK,), jnp.int32),
            pltpu.VMEM((_CHUNK,), jnp.float32),
            pltpu.VMEM((_CHUNK,), jnp.int32),
            pltpu.VMEM((_CHUNK,), jnp.float32),
            pltpu.VMEM((_N_ACC,), jnp.float32),
            pltpu.SemaphoreType.DMA,
            pltpu.SemaphoreType.DMA,
        ],
    )(idx_flat, val)

    summed = pl.pallas_call(
        _tc_reduce_kernel,
        out_shape=jax.ShapeDtypeStruct((_N_ACC,), jnp.float32),
    )(partials)

    return summed.reshape(_N_NODES, 1)


# speedup vs baseline: 1.5084x; 1.5084x over previous
import math

import jax
import jax.numpy as jnp
from jax import lax
from jax.experimental import pallas as pl
from jax.experimental.pallas import tpu as pltpu
from jax.experimental.pallas import tpu_sc as plsc

_N_NODES = 100_000
_N_EDGES = 6_400_000
_FACTOR = 1.0 / math.sqrt(64.0)

_NC = 2
_NS = 16
_NW = _NC * _NS
_EPW = _N_EDGES // _NW
_CHUNK = 4_000
_NCHUNK = _EPW // _CHUNK
_LANES = 16
_N_ACC = 100_352
_UNROLL = 10
_ZUNROLL = 8


def _sc_scatter_phase(idx_hbm, val_hbm, out_hbm,
                      idxb0, valb0, idxb1, valb1, acc, sem0, sem1):
    wid = lax.axis_index("s") * _NC + lax.axis_index("c")
    base = wid * _EPW

    idxbs = (idxb0, idxb1)
    valbs = (valb0, valb1)
    sems = (sem0, sem1)

    def start(c, b):
        off = base + c * _CHUNK
        pltpu.async_copy(idx_hbm.at[pl.ds(off, _CHUNK)], idxbs[b], sems[b])
        pltpu.async_copy(val_hbm.at[pl.ds(off, _CHUNK)], valbs[b], sems[b])

    def wait(b):
        pltpu.make_async_copy(idx_hbm.at[pl.ds(0, _CHUNK)], idxbs[b],
                              sems[b]).wait()
        pltpu.make_async_copy(val_hbm.at[pl.ds(0, _CHUNK)], valbs[b],
                              sems[b]).wait()

    start(0, 0)

    zeros = jnp.zeros((_LANES,), jnp.float32)

    @plsc.parallel_loop(0, _N_ACC // _LANES, unroll=_ZUNROLL)
    def _zero(i):
        acc[pl.ds(i * _LANES, _LANES)] = zeros

    def process(b):
        ib, vb = idxbs[b], valbs[b]

        @plsc.parallel_loop(0, _CHUNK // _LANES, unroll=_UNROLL)
        def _scat(j):
            o = j * _LANES
            iv = ib[pl.ds(o, _LANES)]
            vv = vb[pl.ds(o, _LANES)]
            plsc.addupdate_scatter(acc, [iv], vv)

    def cbody(s, carry):
        c0 = 2 * s
        start(c0 + 1, 1)
        wait(0)
        process(0)
        @pl.when(c0 + 2 < _NCHUNK)
        def _():
            start(c0 + 2, 0)
        wait(1)
        process(1)
        return carry

    lax.fori_loop(0, _NCHUNK // 2, cbody, 0)

    pltpu.sync_copy(acc, out_hbm.at[wid])


def _tc_reduce_kernel(p_ref, o_ref):
    o_ref[...] = jnp.sum(p_ref[...], axis=0) * _FACTOR


@jax.jit
def kernel(edge_features, edge_index, atom_types):
    del atom_types
    idx = edge_index[0]
    val = edge_features.reshape(_N_EDGES)

    mesh = plsc.VectorSubcoreMesh(
        core_axis_name="c", subcore_axis_name="s",
        num_cores=_NC, num_subcores=_NS,
    )
    partials = pl.kernel(
        _sc_scatter_phase,
        out_type=jax.ShapeDtypeStruct((_NW, _N_ACC), jnp.float32),
        mesh=mesh,
        compiler_params=pltpu.CompilerParams(needs_layout_passes=False),
        scratch_types=[
            pltpu.VMEM((_CHUNK,), jnp.int32),
            pltpu.VMEM((_CHUNK,), jnp.float32),
            pltpu.VMEM((_CHUNK,), jnp.int32),
            pltpu.VMEM((_CHUNK,), jnp.float32),
            pltpu.VMEM((_N_ACC,), jnp.float32),
            pltpu.SemaphoreType.DMA,
            pltpu.SemaphoreType.DMA,
        ],
    )(idx, val)

    bw = 14_336
    summed = pl.pallas_call(
        _tc_reduce_kernel,
        grid=(_N_ACC // bw,),
        in_specs=[pl.BlockSpec((_NW, bw), lambda i: (0, i))],
        out_specs=pl.BlockSpec((bw,), lambda i: (i,)),
        out_shape=jax.ShapeDtypeStruct((_N_ACC,), jnp.float32),
    )(partials)

    return summed[:_N_NODES].reshape(_N_NODES, 1)


# speedup vs baseline: 1.7813x; 1.1809x over previous
import math

import jax
import jax.numpy as jnp
from jax import lax
from jax.experimental import pallas as pl
from jax.experimental.pallas import tpu as pltpu
from jax.experimental.pallas import tpu_sc as plsc

_N_NODES = 100_000
_N_EDGES = 6_400_000
_FACTOR = 1.0 / math.sqrt(64.0)

_NC = 2
_NS = 16
_NW = _NC * _NS
_EPW = _N_EDGES // _NW
_CHUNK = 4_000
_NCHUNK = _EPW // _CHUNK
_LANES = 16
_N_ACC = 100_352
_IBUF = 4_096
_UNROLL = 10
_ZUNROLL = 8


def _sc_scatter_phase(idx_hbm, val_hbm, out_hbm,
                      idxb0, valb0, idxb1, valb1, acc, sem0, sem1):
    wid = lax.axis_index("s") * _NC + lax.axis_index("c")
    base = wid * _EPW

    idxbs = (idxb0, idxb1)
    valbs = (valb0, valb1)
    sems = (sem0, sem1)

    def start(c, b):
        off = base + c * _CHUNK
        aoff = pl.multiple_of((off // 128) * 128, 128)
        pltpu.async_copy(idx_hbm.at[:, pl.ds(aoff, _IBUF)], idxbs[b], sems[b])
        pltpu.async_copy(val_hbm.at[pl.ds(off, _CHUNK)], valbs[b], sems[b])

    def wait(b):
        pltpu.make_async_copy(idx_hbm.at[:, pl.ds(0, _IBUF)], idxbs[b],
                              sems[b]).wait()
        pltpu.make_async_copy(val_hbm.at[pl.ds(0, _CHUNK)], valbs[b],
                              sems[b]).wait()

    start(0, 0)

    zeros = jnp.zeros((_LANES,), jnp.float32)

    @plsc.parallel_loop(0, _N_ACC // _LANES, unroll=_ZUNROLL)
    def _zero(i):
        acc[pl.ds(i * _LANES, _LANES)] = zeros

    def process(c, b):
        ib, vb = idxbs[b], valbs[b]
        off = base + c * _CHUNK
        ph = off - (off // 128) * 128

        @plsc.parallel_loop(0, _CHUNK // _LANES, unroll=_UNROLL)
        def _scat(j):
            o = j * _LANES
            iv = ib[0, pl.ds(ph + o, _LANES)]
            vv = vb[pl.ds(o, _LANES)]
            plsc.addupdate_scatter(acc, [iv], vv)

    def cbody(s, carry):
        c0 = 2 * s
        start(c0 + 1, 1)
        wait(0)
        process(c0, 0)
        @pl.when(c0 + 2 < _NCHUNK)
        def _():
            start(c0 + 2, 0)
        wait(1)
        process(c0 + 1, 1)
        return carry

    lax.fori_loop(0, _NCHUNK // 2, cbody, 0)

    pltpu.sync_copy(acc, out_hbm.at[wid])


def _tc_reduce_kernel(p_ref, o_ref):
    o_ref[...] = jnp.sum(p_ref[...], axis=0) * _FACTOR


@jax.jit
def kernel(edge_features, edge_index, atom_types):
    del atom_types
    val = edge_features.reshape(_N_EDGES)

    mesh = plsc.VectorSubcoreMesh(
        core_axis_name="c", subcore_axis_name="s",
        num_cores=_NC, num_subcores=_NS,
    )
    partials = pl.kernel(
        _sc_scatter_phase,
        out_type=jax.ShapeDtypeStruct((_NW, _N_ACC), jnp.float32),
        mesh=mesh,
        compiler_params=pltpu.CompilerParams(needs_layout_passes=False),
        scratch_types=[
            pltpu.VMEM((2, _IBUF), jnp.int32),
            pltpu.VMEM((_CHUNK,), jnp.float32),
            pltpu.VMEM((2, _IBUF), jnp.int32),
            pltpu.VMEM((_CHUNK,), jnp.float32),
            pltpu.VMEM((_N_ACC,), jnp.float32),
            pltpu.SemaphoreType.DMA,
            pltpu.SemaphoreType.DMA,
        ],
    )(edge_index, val)

    bw = 14_336
    summed = pl.pallas_call(
        _tc_reduce_kernel,
        grid=(_N_ACC // bw,),
        in_specs=[pl.BlockSpec((_NW, bw), lambda i: (0, i))],
        out_specs=pl.BlockSpec((bw,), lambda i: (i,)),
        out_shape=jax.ShapeDtypeStruct((_N_ACC,), jnp.float32),
    )(partials)

    return summed[:_N_NODES].reshape(_N_NODES, 1)
